# baseline (speedup 1.0000x reference)
import jax
import jax.numpy as jnp
from jax import lax
from jax.experimental import pallas as pl
from jax.experimental.pallas import tpu as pltpu
from jax.experimental.pallas import tpu_sc as plsc

N_NODES = 10000
N_EDGES = 320000
D = 128
HALF = D // 2

NUM_CORES = 2
NUM_SUBCORES = 16
EDGES_PER_TILE = N_EDGES // NUM_SUBCORES
CHUNK = 160
NCHUNKS = EDGES_PER_TILE // CHUNK
ROWS_PER_TILE = 624
TAIL_BASE = ROWS_PER_TILE * NUM_SUBCORES
TAIL = N_NODES - TAIL_BASE


NB = 4
GD = 2


def _sc_body(feat3_h, src2_h, dst2_h, nout_h, dout_h,
             degl, idx_s, idx_d, r0, r1, r2, r3, nacc,
             g0, g1, g2, g3, s0, s1, s2, s3, isem):
    rowbufs = (r0, r1, r2, r3)
    gsems = (g0, g1, g2, g3)
    scsems = (s0, s1, s2, s3)
    c = lax.axis_index("c")
    s = lax.axis_index("s")

    z16 = jnp.zeros((16,), jnp.float32)
    ones16 = jnp.ones((16,), jnp.float32)

    ixs = pltpu.async_copy(src2_h.at[pl.ds(s * NCHUNKS, NCHUNKS)], idx_s, isem)
    ixd = pltpu.async_copy(dst2_h.at[pl.ds(s * NCHUNKS, NCHUNKS)], idx_d, isem)

    def zero_zbuf(i, carry):
        for k in range(HALF // 16):
            r0[i, pl.ds(k * 16, 16)] = z16
        return carry

    lax.fori_loop(0, CHUNK, zero_zbuf, 0)

    def zero_deg(i, carry):
        degl[pl.ds(i * 16, 16)] = z16
        return carry

    lax.fori_loop(0, N_NODES // 16, zero_deg, 0)

    for q in range(ROWS_PER_TILE // CHUNK):
        pltpu.sync_copy(r0, nacc.at[pl.ds(s * ROWS_PER_TILE + q * CHUNK, CHUNK)])
    rem = ROWS_PER_TILE % CHUNK
    if rem:
        pltpu.sync_copy(r0.at[pl.ds(0, rem)],
                        nacc.at[pl.ds(s * ROWS_PER_TILE + ROWS_PER_TILE - rem, rem)])

    @pl.when(s == NUM_SUBCORES - 1)
    def _():
        pltpu.sync_copy(r0.at[pl.ds(0, TAIL)], nacc.at[pl.ds(TAIL_BASE, TAIL)])

    ixs.wait()
    ixd.wait()
    plsc.subcore_barrier()

    feat_c = feat3_h.at[c]

    def _gather(j, b):
        return pltpu.make_async_copy(feat_c.at[idx_s.at[j]], rowbufs[b],
                                     gsems[b])

    def _scatter(j, b):
        return pltpu.make_async_copy(rowbufs[b], nacc.at[idx_d.at[j]],
                                     scsems[b])

    def pipe(i, carry):
        for b in range(NB):
            j = NB * i + b

            @pl.when(j < NCHUNKS)
            def _():
                @pl.when(j >= NB)
                def _():
                    _scatter(j - NB, b).wait()

                _gather(j, b).start()

                @pl.when(c == 0)
                def _():
                    for k in range(CHUNK // 16):
                        i16 = idx_d[j, pl.ds(k * 16, 16)]
                        plsc.addupdate_scatter(degl, [i16], ones16)

            pb = (b - GD) % NB

            @pl.when(jnp.logical_and(j >= GD, j < NCHUNKS + GD))
            def _():
                p = j - GD
                _gather(p, pb).wait()
                _scatter(p, pb).start(add=True)

        return carry

    lax.fori_loop(0, (NCHUNKS + GD + NB - 1) // NB, pipe, 0)

    for t in range(NB):
        q = NCHUNKS - NB + t
        _scatter(q, q % NB).wait()
    plsc.subcore_barrier()

    nout_c = nout_h.at[:, pl.ds(c * HALF, HALF)]
    pltpu.sync_copy(nacc.at[pl.ds(s * ROWS_PER_TILE, ROWS_PER_TILE)],
                    nout_c.at[pl.ds(s * ROWS_PER_TILE, ROWS_PER_TILE)])

    @pl.when(s == NUM_SUBCORES - 1)
    def _():
        pltpu.sync_copy(nacc.at[pl.ds(TAIL_BASE, TAIL)],
                        nout_c.at[pl.ds(TAIL_BASE, TAIL)])

    @pl.when(c == 0)
    def _():
        pltpu.sync_copy(degl, dout_h.at[s, 0])


def _sc_aggregate(feat3, src2, dst2):
    mesh = plsc.VectorSubcoreMesh(core_axis_name="c", subcore_axis_name="s")
    f = pl.kernel(
        _sc_body,
        out_type=[
            jax.ShapeDtypeStruct((N_NODES, D), jnp.float32),
            jax.ShapeDtypeStruct((NUM_SUBCORES, 1, N_NODES), jnp.float32),
        ],
        mesh=mesh,
        compiler_params=pltpu.CompilerParams(
            needs_layout_passes=False, use_tc_tiling_on_sc=False),
        scratch_types=[
            pltpu.VMEM((N_NODES,), jnp.float32),
            pltpu.VMEM((NCHUNKS, CHUNK), jnp.int32),
            pltpu.VMEM((NCHUNKS, CHUNK), jnp.int32),
        ] + [pltpu.VMEM((CHUNK, HALF), jnp.float32) for _ in range(NB)]
        + [pltpu.VMEM_SHARED((N_NODES, HALF), jnp.float32)]
        + [pltpu.SemaphoreType.DMA for _ in range(2 * NB + 1)],
    )
    return f(feat3, src2, dst2)


def _tc_body(f_ref, n_ref, dp_ref, w_ref, b_ref, o_ref):
    rec = 1.0 / jnp.maximum(dp_ref[...], 1.0)
    hk = n_ref[...] * rec
    w = w_ref[...]
    dn = (((1,), (1,)), ((), ()))
    acc = lax.dot_general(f_ref[...], w[:, :D], dn,
                          preferred_element_type=jnp.float32)
    acc = acc + lax.dot_general(hk, w[:, D:], dn,
                                preferred_element_type=jnp.float32)
    o_ref[...] = jnp.maximum(acc + b_ref[...], 0.0)


BLK = 1000


def _tc_dense(feature, nsum, dcol, W, brow):
    grid = (N_NODES // BLK,)
    return pl.pallas_call(
        _tc_body,
        grid=grid,
        in_specs=[
            pl.BlockSpec((BLK, D), lambda i: (i, 0)),
            pl.BlockSpec((BLK, D), lambda i: (i, 0)),
            pl.BlockSpec((BLK, 1), lambda i: (i, 0)),
            pl.BlockSpec((D, 2 * D), lambda i: (0, 0)),
            pl.BlockSpec((1, D), lambda i: (0, 0)),
        ],
        out_specs=pl.BlockSpec((BLK, D), lambda i: (i, 0)),
        out_shape=jax.ShapeDtypeStruct((N_NODES, D), jnp.float32),
    )(feature, nsum, dcol, W, brow)


def kernel(feature, edge_index, W, b):
    src2 = edge_index[0].astype(jnp.int32).reshape(N_EDGES // CHUNK, CHUNK)
    dst2 = edge_index[1].astype(jnp.int32).reshape(N_EDGES // CHUNK, CHUNK)
    feat3 = jnp.stack([feature[:, :HALF], feature[:, HALF:]])
    nsum, dhist = _sc_aggregate(feat3, src2, dst2)
    dcol = dhist.reshape(NUM_SUBCORES, N_NODES).sum(axis=0).reshape(N_NODES, 1)
    return _tc_dense(feature, nsum, dcol, W, b.reshape(1, D))


# speedup vs baseline: 1.0530x; 1.0530x over previous
import jax
import jax.numpy as jnp
from jax import lax
from jax.experimental import pallas as pl
from jax.experimental.pallas import tpu as pltpu
from jax.experimental.pallas import tpu_sc as plsc

N_NODES = 10000
N_EDGES = 320000
D = 128
HALF = D // 2

NUM_CORES = 2
NUM_SUBCORES = 16
EDGES_PER_TILE = N_EDGES // NUM_SUBCORES
CHUNK = 80
NCHUNKS = EDGES_PER_TILE // CHUNK
ROWS_PER_TILE = 624
TAIL_BASE = ROWS_PER_TILE * NUM_SUBCORES
TAIL = N_NODES - TAIL_BASE


NB = 6
GD = 3


def _sc_body(feat3_h, src2_h, dst2_h, nout_h, dout_h,
             degl, idx_s, idx_d, r0, r1, r2, r3, r4, r5, nacc,
             g0, g1, g2, g3, g4, g5, s0, s1, s2, s3, s4, s5, isem):
    rowbufs = (r0, r1, r2, r3, r4, r5)
    gsems = (g0, g1, g2, g3, g4, g5)
    scsems = (s0, s1, s2, s3, s4, s5)
    c = lax.axis_index("c")
    s = lax.axis_index("s")

    z16 = jnp.zeros((16,), jnp.float32)
    ones16 = jnp.ones((16,), jnp.float32)

    ixs = pltpu.async_copy(src2_h.at[pl.ds(s * NCHUNKS, NCHUNKS)], idx_s, isem)
    ixd = pltpu.async_copy(dst2_h.at[pl.ds(s * NCHUNKS, NCHUNKS)], idx_d, isem)

    def zero_zbuf(i, carry):
        for k in range(HALF // 16):
            r0[i, pl.ds(k * 16, 16)] = z16
        return carry

    lax.fori_loop(0, CHUNK, zero_zbuf, 0)

    def zero_deg(i, carry):
        degl[pl.ds(i * 16, 16)] = z16
        return carry

    lax.fori_loop(0, N_NODES // 16, zero_deg, 0)

    for q in range(ROWS_PER_TILE // CHUNK):
        pltpu.sync_copy(r0, nacc.at[pl.ds(s * ROWS_PER_TILE + q * CHUNK, CHUNK)])
    rem = ROWS_PER_TILE % CHUNK
    if rem:
        pltpu.sync_copy(r0.at[pl.ds(0, rem)],
                        nacc.at[pl.ds(s * ROWS_PER_TILE + ROWS_PER_TILE - rem, rem)])

    @pl.when(s == NUM_SUBCORES - 1)
    def _():
        pltpu.sync_copy(r0.at[pl.ds(0, TAIL)], nacc.at[pl.ds(TAIL_BASE, TAIL)])

    ixs.wait()
    ixd.wait()
    plsc.subcore_barrier()

    feat_c = feat3_h.at[c]

    def _gather(j, b):
        return pltpu.make_async_copy(feat_c.at[idx_s.at[j]], rowbufs[b],
                                     gsems[b])

    def _scatter(j, b):
        return pltpu.make_async_copy(rowbufs[b], nacc.at[idx_d.at[j]],
                                     scsems[b])

    def pipe(i, carry):
        for b in range(NB):
            j = NB * i + b

            @pl.when(j < NCHUNKS)
            def _():
                @pl.when(j >= NB)
                def _():
                    _scatter(j - NB, b).wait()

                _gather(j, b).start()

                @pl.when(c == 0)
                def _():
                    for k in range(CHUNK // 16):
                        i16 = idx_d[j, pl.ds(k * 16, 16)]
                        plsc.addupdate_scatter(degl, [i16], ones16)

            pb = (b - GD) % NB

            @pl.when(jnp.logical_and(j >= GD, j < NCHUNKS + GD))
            def _():
                p = j - GD
                _gather(p, pb).wait()
                _scatter(p, pb).start(add=True)

        return carry

    lax.fori_loop(0, (NCHUNKS + GD + NB - 1) // NB, pipe, 0)

    for t in range(NB):
        q = NCHUNKS - NB + t
        _scatter(q, q % NB).wait()
    plsc.subcore_barrier()

    nout_c = nout_h.at[:, pl.ds(c * HALF, HALF)]
    pltpu.sync_copy(nacc.at[pl.ds(s * ROWS_PER_TILE, ROWS_PER_TILE)],
                    nout_c.at[pl.ds(s * ROWS_PER_TILE, ROWS_PER_TILE)])

    @pl.when(s == NUM_SUBCORES - 1)
    def _():
        pltpu.sync_copy(nacc.at[pl.ds(TAIL_BASE, TAIL)],
                        nout_c.at[pl.ds(TAIL_BASE, TAIL)])

    @pl.when(c == 0)
    def _():
        pltpu.sync_copy(degl, dout_h.at[s])


def _sc_aggregate(feat3, src2, dst2):
    mesh = plsc.VectorSubcoreMesh(core_axis_name="c", subcore_axis_name="s")
    f = pl.kernel(
        _sc_body,
        out_type=[
            jax.ShapeDtypeStruct((N_NODES, D), jnp.float32),
            jax.ShapeDtypeStruct((NUM_SUBCORES, N_NODES), jnp.float32),
        ],
        mesh=mesh,
        compiler_params=pltpu.CompilerParams(
            needs_layout_passes=False, use_tc_tiling_on_sc=False),
        scratch_types=[
            pltpu.VMEM((N_NODES,), jnp.float32),
            pltpu.VMEM((NCHUNKS, CHUNK), jnp.int32),
            pltpu.VMEM((NCHUNKS, CHUNK), jnp.int32),
        ] + [pltpu.VMEM((CHUNK, HALF), jnp.float32) for _ in range(NB)]
        + [pltpu.VMEM_SHARED((N_NODES, HALF), jnp.float32)]
        + [pltpu.SemaphoreType.DMA for _ in range(2 * NB + 1)],
    )
    return f(feat3, src2, dst2)


def _tc_body(f_ref, n_ref, dp_ref, w_ref, b_ref, o_ref):
    deg = lax.dot_general(dp_ref[...], jnp.ones((NUM_SUBCORES, 1), jnp.float32),
                          (((1,), (0,)), ((), ())),
                          preferred_element_type=jnp.float32)
    rec = 1.0 / jnp.maximum(deg, 1.0)
    hk = n_ref[...] * rec
    w = w_ref[...]
    dn = (((1,), (1,)), ((), ()))
    acc = lax.dot_general(f_ref[...], w[:, :D], dn,
                          preferred_element_type=jnp.float32)
    acc = acc + lax.dot_general(hk, w[:, D:], dn,
                                preferred_element_type=jnp.float32)
    o_ref[...] = jnp.maximum(acc + b_ref[...], 0.0)


BLK = 1000


def _tc_dense(feature, nsum, dhist, W, brow):
    grid = (N_NODES // BLK,)
    return pl.pallas_call(
        _tc_body,
        grid=grid,
        in_specs=[
            pl.BlockSpec((BLK, D), lambda i: (i, 0)),
            pl.BlockSpec((BLK, D), lambda i: (i, 0)),
            pl.BlockSpec((BLK, NUM_SUBCORES), lambda i: (i, 0)),
            pl.BlockSpec((D, 2 * D), lambda i: (0, 0)),
            pl.BlockSpec((1, D), lambda i: (0, 0)),
        ],
        out_specs=pl.BlockSpec((BLK, D), lambda i: (i, 0)),
        out_shape=jax.ShapeDtypeStruct((N_NODES, D), jnp.float32),
    )(feature, nsum, dhist, W, brow)


def kernel(feature, edge_index, W, b):
    src2 = edge_index[0].astype(jnp.int32).reshape(N_EDGES // CHUNK, CHUNK)
    dst2 = edge_index[1].astype(jnp.int32).reshape(N_EDGES // CHUNK, CHUNK)
    feat3 = jnp.stack([feature[:, :HALF], feature[:, HALF:]])
    nsum, dhist = _sc_aggregate(feat3, src2, dst2)
    return _tc_dense(feature, nsum, dhist.T, W, b.reshape(1, D))


# speedup vs baseline: 1.1593x; 1.1009x over previous
import jax
import jax.numpy as jnp
from jax import lax
from jax.experimental import pallas as pl
from jax.experimental.pallas import tpu as pltpu
from jax.experimental.pallas import tpu_sc as plsc

N_NODES = 10000
N_EDGES = 320000
D = 128
HALF = D // 2

NUM_CORES = 2
NUM_SUBCORES = 16
EDGES_PER_TILE = N_EDGES // NUM_SUBCORES
CHUNK = 80
NCHUNKS = EDGES_PER_TILE // CHUNK
ROWS_PER_TILE = 624
TAIL_BASE = ROWS_PER_TILE * NUM_SUBCORES
TAIL = N_NODES - TAIL_BASE


NB = 6
GD = 3


def _sc_body(feat2n_h, src2x_h, dst1_h, nout_h, dout_h,
             degl, idx_s, idx_d, r0, r1, r2, r3, r4, r5, nacc,
             g0, g1, g2, g3, g4, g5, s0, s1, s2, s3, s4, s5, isem):
    rowbufs = (r0, r1, r2, r3, r4, r5)
    gsems = (g0, g1, g2, g3, g4, g5)
    scsems = (s0, s1, s2, s3, s4, s5)
    c = lax.axis_index("c")
    s = lax.axis_index("s")

    z16 = jnp.zeros((16,), jnp.float32)
    ones16 = jnp.ones((16,), jnp.float32)

    ixs = pltpu.async_copy(src2x_h.at[pl.ds(s * EDGES_PER_TILE, EDGES_PER_TILE)],
                           idx_s, isem)
    ixd = pltpu.async_copy(dst1_h.at[pl.ds(s * EDGES_PER_TILE, EDGES_PER_TILE)],
                           idx_d, isem)

    def zero_zbuf(i, carry):
        for k in range(HALF // 16):
            r0[i, pl.ds(k * 16, 16)] = z16
        return carry

    lax.fori_loop(0, CHUNK, zero_zbuf, 0)

    def zero_deg(i, carry):
        degl[pl.ds(i * 16, 16)] = z16
        return carry

    lax.fori_loop(0, N_NODES // 16, zero_deg, 0)

    for q in range(ROWS_PER_TILE // CHUNK):
        pltpu.sync_copy(r0, nacc.at[pl.ds(s * ROWS_PER_TILE + q * CHUNK, CHUNK)])
    rem = ROWS_PER_TILE % CHUNK
    if rem:
        pltpu.sync_copy(r0.at[pl.ds(0, rem)],
                        nacc.at[pl.ds(s * ROWS_PER_TILE + ROWS_PER_TILE - rem, rem)])

    @pl.when(s == NUM_SUBCORES - 1)
    def _():
        pltpu.sync_copy(r0.at[pl.ds(0, TAIL)], nacc.at[pl.ds(TAIL_BASE, TAIL)])

    ixs.wait()
    ixd.wait()
    plsc.subcore_barrier()

    feat_c = feat2n_h.at[pl.ds(c, 2 * N_NODES - 1)]

    def _gather(j, b):
        return pltpu.make_async_copy(
            feat_c.at[idx_s.at[pl.ds(j * CHUNK, CHUNK)]], rowbufs[b], gsems[b])

    def _scatter(j, b):
        return pltpu.make_async_copy(
            rowbufs[b], nacc.at[idx_d.at[pl.ds(j * CHUNK, CHUNK)]], scsems[b])

    def pipe(i, carry):
        for b in range(NB):
            j = NB * i + b

            @pl.when(j < NCHUNKS)
            def _():
                @pl.when(j >= NB)
                def _():
                    _scatter(j - NB, b).wait()

                _gather(j, b).start()

                @pl.when(c == 0)
                def _():
                    for k in range(CHUNK // 16):
                        i16 = idx_d[pl.ds(j * CHUNK + k * 16, 16)]
                        plsc.addupdate_scatter(degl, [i16], ones16)

            pb = (b - GD) % NB

            @pl.when(jnp.logical_and(j >= GD, j < NCHUNKS + GD))
            def _():
                p = j - GD
                _gather(p, pb).wait()
                _scatter(p, pb).start(add=True)

        return carry

    lax.fori_loop(0, (NCHUNKS + GD + NB - 1) // NB, pipe, 0)

    for t in range(NB):
        q = NCHUNKS - NB + t
        _scatter(q, q % NB).wait()
    plsc.subcore_barrier()

    nout_c = nout_h.at[:, pl.ds(c * HALF, HALF)]
    pltpu.sync_copy(nacc.at[pl.ds(s * ROWS_PER_TILE, ROWS_PER_TILE)],
                    nout_c.at[pl.ds(s * ROWS_PER_TILE, ROWS_PER_TILE)])

    @pl.when(s == NUM_SUBCORES - 1)
    def _():
        pltpu.sync_copy(nacc.at[pl.ds(TAIL_BASE, TAIL)],
                        nout_c.at[pl.ds(TAIL_BASE, TAIL)])

    @pl.when(c == 0)
    def _():
        pltpu.sync_copy(degl, dout_h.at[s])


def _sc_aggregate(feat2n, src2x, dst1):
    mesh = plsc.VectorSubcoreMesh(core_axis_name="c", subcore_axis_name="s")
    f = pl.kernel(
        _sc_body,
        out_type=[
            jax.ShapeDtypeStruct((N_NODES, D), jnp.float32),
            jax.ShapeDtypeStruct((NUM_SUBCORES, N_NODES), jnp.float32),
        ],
        mesh=mesh,
        compiler_params=pltpu.CompilerParams(
            needs_layout_passes=False, use_tc_tiling_on_sc=False),
        scratch_types=[
            pltpu.VMEM((N_NODES,), jnp.float32),
            pltpu.VMEM((EDGES_PER_TILE,), jnp.int32),
            pltpu.VMEM((EDGES_PER_TILE,), jnp.int32),
        ] + [pltpu.VMEM((CHUNK, HALF), jnp.float32) for _ in range(NB)]
        + [pltpu.VMEM_SHARED((N_NODES, HALF), jnp.float32)]
        + [pltpu.SemaphoreType.DMA for _ in range(2 * NB + 1)],
    )
    return f(feat2n, src2x, dst1)


def _tc_body(f_ref, n_ref, dp_ref, w_ref, b_ref, o_ref):
    deg = lax.dot_general(dp_ref[...], jnp.ones((NUM_SUBCORES, 1), jnp.float32),
                          (((1,), (0,)), ((), ())),
                          preferred_element_type=jnp.float32)
    rec = 1.0 / jnp.maximum(deg, 1.0)
    hk = n_ref[...] * rec
    w = w_ref[...]
    dn = (((1,), (1,)), ((), ()))
    acc = lax.dot_general(f_ref[...], w[:, :D], dn,
                          preferred_element_type=jnp.float32)
    acc = acc + lax.dot_general(hk, w[:, D:], dn,
                                preferred_element_type=jnp.float32)
    o_ref[...] = jnp.maximum(acc + b_ref[...], 0.0)


BLK = 1000


def _tc_dense(feature, nsum, dhist, W, brow):
    grid = (N_NODES // BLK,)
    return pl.pallas_call(
        _tc_body,
        grid=grid,
        in_specs=[
            pl.BlockSpec((BLK, D), lambda i: (i, 0)),
            pl.BlockSpec((BLK, D), lambda i: (i, 0)),
            pl.BlockSpec((BLK, NUM_SUBCORES), lambda i: (i, 0)),
            pl.BlockSpec((D, 2 * D), lambda i: (0, 0)),
            pl.BlockSpec((1, D), lambda i: (0, 0)),
        ],
        out_specs=pl.BlockSpec((BLK, D), lambda i: (i, 0)),
        out_shape=jax.ShapeDtypeStruct((N_NODES, D), jnp.float32),
    )(feature, nsum, dhist, W, brow)


def kernel(feature, edge_index, W, b):
    ei = edge_index.astype(jnp.int32)
    src2x = ei[0] * 2
    dst1 = ei[1]
    feat2n = feature.reshape(2 * N_NODES, HALF)
    nsum, dhist = _sc_aggregate(feat2n, src2x, dst1)
    return _tc_dense(feature, nsum, dhist.T, W, b.reshape(1, D))
